# baseline (speedup 1.0000x reference)
import jax
import jax.numpy as jnp
from jax import lax
from jax.experimental import pallas as pl
from jax.experimental.pallas import tpu as pltpu

_N_WAYS = 64
_FEAT = 64
_BK = 8000


def _knn_kernel(xs_ref, c_ref, keys_ref, labels_ref, vals_ref, valid_ref,
                a_ref, a2_ref, min_ref, arg_ref):
    step = pl.program_id(0)
    nsteps = pl.num_programs(0)

    @pl.when(step == 0)
    def _init():
        xs = xs_ref[...]
        keys = jnp.mean(xs.reshape(_N_WAYS, -1, _FEAT), axis=1)
        keys_ref[...] = keys
        a_ref[...] = keys
        a2_ref[...] = jnp.sum(keys * keys, axis=1, keepdims=True)
        min_ref[...] = jnp.full((_N_WAYS, 1), jnp.inf, jnp.float32)
        arg_ref[...] = jnp.zeros((_N_WAYS, 1), jnp.int32)

    a = a_ref[...]
    b = c_ref[...]
    ab = lax.dot_general(a, b, (((1,), (1,)), ((), ())),
                         preferred_element_type=jnp.float32)
    ones = jnp.ones((1, _FEAT), jnp.float32)
    b2 = lax.dot_general(ones, b * b, (((1,), (1,)), ((), ())),
                         preferred_element_type=jnp.float32)
    d2 = jnp.maximum(a2_ref[...] + b2 - 2.0 * ab, 1e-12)
    bmin = jnp.min(d2, axis=1, keepdims=True)
    idx = lax.broadcasted_iota(jnp.int32, d2.shape, 1) + step * _BK
    barg = jnp.min(jnp.where(d2 == bmin, idx, jnp.int32(2**31 - 1)),
                   axis=1, keepdims=True)
    better = bmin < min_ref[...]
    arg_ref[...] = jnp.where(better, barg, arg_ref[...])
    min_ref[...] = jnp.where(better, bmin, min_ref[...])

    @pl.when(step == nsteps - 1)
    def _fin():
        vals_ref[...] = jnp.sqrt(min_ref[...])
        labels_ref[...] = arg_ref[...]
        lab_col = arg_ref[...]
        lab_row = lab_col.reshape(1, _N_WAYS)
        n_eq = jnp.sum((lab_col == lab_row).astype(jnp.int32))
        valid_ref[...] = jnp.full((8, 128), (n_eq == _N_WAYS).astype(jnp.int32))


def kernel(combined_xs, centroid):
    k_rows = centroid.shape[0]
    nsteps = k_rows // _BK
    class_keys, labels, vals, valid = pl.pallas_call(
        _knn_kernel,
        grid=(nsteps,),
        in_specs=[
            pl.BlockSpec(combined_xs.shape, lambda i: (0, 0)),
            pl.BlockSpec((_BK, _FEAT), lambda i: (i, 0)),
        ],
        out_specs=[
            pl.BlockSpec((_N_WAYS, _FEAT), lambda i: (0, 0)),
            pl.BlockSpec((_N_WAYS, 1), lambda i: (0, 0)),
            pl.BlockSpec((_N_WAYS, 1), lambda i: (0, 0)),
            pl.BlockSpec((8, 128), lambda i: (0, 0)),
        ],
        out_shape=[
            jax.ShapeDtypeStruct((_N_WAYS, _FEAT), jnp.float32),
            jax.ShapeDtypeStruct((_N_WAYS, 1), jnp.int32),
            jax.ShapeDtypeStruct((_N_WAYS, 1), jnp.float32),
            jax.ShapeDtypeStruct((8, 128), jnp.int32),
        ],
        scratch_shapes=[
            pltpu.VMEM((_N_WAYS, _FEAT), jnp.float32),
            pltpu.VMEM((_N_WAYS, 1), jnp.float32),
            pltpu.VMEM((_N_WAYS, 1), jnp.float32),
            pltpu.VMEM((_N_WAYS, 1), jnp.int32),
        ],
    )(combined_xs, centroid)
    return (class_keys, labels.reshape(-1), vals, valid[0, 0] != 0)
